# Initial kernel scaffold; baseline (speedup 1.0000x reference)
import jax
import jax.numpy as jnp
from jax.experimental import pallas as pl


def kernel(x, edge_index, W0i, as0i, ad0i, b0i, W0o, as0o, ad0o, b0o, W1i, as1i, ad1i, b1i, W1o, as1o, ad1o, b1o):
    raise NotImplementedError("write your pallas kernel here")


# speedup vs baseline: 1.0206x; 1.0206x over previous
import functools

import jax
import jax.numpy as jnp
from jax.experimental import pallas as pl

N = 10000
E = 320000
IN_CH = 128
HID = 256
HEADS = 4


def _linear_body(x_ref, w_ref, out_ref):
    out_ref[:] = jnp.dot(x_ref[:], w_ref[:], preferred_element_type=jnp.float32)


def _linear(x, w, block_rows=1000):
    n, k = x.shape
    m = w.shape[1]
    return pl.pallas_call(
        _linear_body,
        grid=(n // block_rows,),
        in_specs=[
            pl.BlockSpec((block_rows, k), lambda i: (i, 0)),
            pl.BlockSpec((k, m), lambda i: (0, 0)),
        ],
        out_specs=pl.BlockSpec((block_rows, m), lambda i: (i, 0)),
        out_shape=jax.ShapeDtypeStruct((n, m), jnp.float32),
    )(x, w)


def _gat_conv(x, s, d, W, att_src, att_dst, bias, heads, out_ch):
    loop = jnp.arange(N, dtype=s.dtype)
    s = jnp.concatenate([s, loop])
    d = jnp.concatenate([d, loop])
    h = _linear(x, W).reshape(N, heads, out_ch)
    a_s = jnp.sum(h * att_src[None, :, :], axis=-1)
    a_d = jnp.sum(h * att_dst[None, :, :], axis=-1)
    alpha = a_s[s] + a_d[d]
    alpha = jax.nn.leaky_relu(alpha, negative_slope=0.2)
    amax = jax.ops.segment_max(alpha, d, num_segments=N)
    alpha = jnp.exp(alpha - amax[d])
    denom = jax.ops.segment_sum(alpha, d, num_segments=N)
    alpha = alpha / (denom[d] + 1e-16)
    msg = h[s] * alpha[:, :, None]
    out = jax.ops.segment_sum(msg, d, num_segments=N)
    return out.reshape(N, heads * out_ch) + bias


def kernel(x, edge_index, W0i, as0i, ad0i, b0i, W0o, as0o, ad0o, b0o,
           W1i, as1i, ad1i, b1i, W1o, as1o, ad1o, b1o):
    row, col = edge_index[0], edge_index[1]
    xi = _gat_conv(x, row, col, W0i, as0i, ad0i, b0i, HEADS, HID // HEADS)
    xo = _gat_conv(x, col, row, W0o, as0o, ad0o, b0o, HEADS, HID // HEADS)
    x1 = jnp.concatenate([xi, xo], axis=1)
    x1 = jax.nn.relu(x1)
    xi2 = _gat_conv(x1, row, col, W1i, as1i, ad1i, b1i, 1, HID)
    xo2 = _gat_conv(x1, col, row, W1o, as1o, ad1o, b1o, 1, HID)
    return jnp.concatenate([xi2, xo2], axis=1)


# speedup vs baseline: 11.8679x; 11.6287x over previous
import functools

import jax
import jax.numpy as jnp
from jax import lax
from jax.experimental import pallas as pl
from jax.experimental.pallas import tpu as pltpu
from jax.experimental.pallas import tpu_sc as plsc

N = 10000
E = 320000
IN_CH = 128
HID = 256
HEADS = 4

NS = 16
BK = 80
NP = 10240
RPT = NP // NS
ET = E // NS
NB = ET // BK


def _sc_den_body(nh, src_hbm, dst_hbm, as_hbm, ad_hbm, w_hbm, den_hbm,
                 asv, adv, denv, idx_s, idx_d, wbuf):
    sid = lax.axis_index("s")
    lanes = lax.iota(jnp.int32, 16)
    zv = jnp.zeros((16,), jnp.float32)

    pltpu.sync_copy(as_hbm, asv)
    pltpu.sync_copy(ad_hbm, adv)

    def _zero_den(k, _):
        denv[pl.ds(k * 16, 16)] = zv
        return 0
    lax.fori_loop(0, NP * nh // 16, _zero_den, 0)

    def _block(b, _):
        base = sid * ET + b * BK
        pltpu.sync_copy(src_hbm.at[pl.ds(base, BK)], idx_s)
        pltpu.sync_copy(dst_hbm.at[pl.ds(base, BK)], idx_d)

        def _grp(g, _):
            sv = idx_s[pl.ds(g * 16, 16)]
            dv = idx_d[pl.ds(g * 16, 16)]
            if nh > 1:
                sv = sv * nh
                dv = dv * nh
            for j in range(nh):
                al = (plsc.load_gather(asv, [sv + j])
                      + plsc.load_gather(adv, [dv + j]))
                w = jnp.exp(jnp.maximum(al, 0.2 * al))
                wbuf[pl.ds(j * BK + g * 16, 16)] = w
                for k16 in range(16):
                    plsc.addupdate_scatter(denv, [dv + j], w,
                                           mask=lanes == k16)
            return 0
        lax.fori_loop(0, BK // 16, _grp, 0)

        for j in range(nh):
            pltpu.sync_copy(wbuf.at[pl.ds(j * BK, BK)],
                            w_hbm.at[pl.ds(j * E + base, BK)])
        return 0
    lax.fori_loop(0, NB, _block, 0)

    pltpu.sync_copy(denv, den_hbm.at[pl.ds(sid * NP * nh, NP * nh)])


@functools.lru_cache(maxsize=None)
def _sc_den_fn(nh):
    mesh = plsc.VectorSubcoreMesh(core_axis_name="c", subcore_axis_name="s",
                                  num_cores=1)
    return pl.kernel(
        functools.partial(_sc_den_body, nh),
        out_type=[
            jax.ShapeDtypeStruct((nh * E,), jnp.float32),
            jax.ShapeDtypeStruct((NS * NP * nh,), jnp.float32),
        ],
        mesh=mesh,
        scratch_types=[
            pltpu.VMEM((N * nh,), jnp.float32),
            pltpu.VMEM((N * nh,), jnp.float32),
            pltpu.VMEM((NP * nh,), jnp.float32),
            pltpu.VMEM((BK,), jnp.int32),
            pltpu.VMEM((BK,), jnp.int32),
            pltpu.VMEM((nh * BK,), jnp.float32),
        ],
        compiler_params=pltpu.CompilerParams(needs_layout_passes=False),
    )


def _sc_num_body(hpc, head_off, src_hbm, dst_hbm, h_hbm, w_hbm, acc_hbm,
                 idx_s, idx_d, wbuf, rows, stag, acc):
    cph = 128 // hpc
    sid = lax.axis_index("s")
    zv = jnp.zeros((16,), jnp.float32)

    def _zero_row(k, _):
        for g in range(128 // 16):
            stag[k, pl.ds(g * 16, 16)] = zv
        return 0
    lax.fori_loop(0, BK, _zero_row, 0)
    for i in range(RPT // BK):
        pltpu.sync_copy(stag, acc.at[pl.ds(sid * RPT + i * BK, BK)])
    plsc.subcore_barrier()

    def _block(b, _):
        base = sid * ET + b * BK
        pltpu.sync_copy(src_hbm.at[pl.ds(base, BK)], idx_s)
        pltpu.sync_copy(dst_hbm.at[pl.ds(base, BK)], idx_d)
        pltpu.sync_copy(h_hbm.at[idx_s], rows)
        for j in range(hpc):
            pltpu.sync_copy(w_hbm.at[pl.ds((head_off + j) * E + base, BK)],
                            wbuf.at[pl.ds(j * BK, BK)])

        def _grp(g, _):
            wvecs = [wbuf[pl.ds(j * BK + g * 16, 16)] for j in range(hpc)]
            for k16 in range(16):
                k = g * 16 + k16
                for j in range(hpc):
                    wv = jnp.full((16,), wvecs[j][k16])
                    for v in range(cph // 16):
                        col = j * cph + v * 16
                        stag[k, pl.ds(col, 16)] = (
                            rows[k, pl.ds(col, 16)] * wv)
            return 0
        lax.fori_loop(0, BK // 16, _grp, 0)

        pltpu.sync_copy(stag, acc.at[idx_d], add=True)
        return 0
    lax.fori_loop(0, NB, _block, 0)

    plsc.subcore_barrier()
    pltpu.sync_copy(acc.at[pl.ds(sid * RPT, RPT)],
                    acc_hbm.at[pl.ds(sid * RPT, RPT)])


@functools.lru_cache(maxsize=None)
def _sc_num_fn(hpc, head_off):
    mesh = plsc.VectorSubcoreMesh(core_axis_name="c", subcore_axis_name="s",
                                  num_cores=1)
    return pl.kernel(
        functools.partial(_sc_num_body, hpc, head_off),
        out_type=jax.ShapeDtypeStruct((NP, 128), jnp.float32),
        mesh=mesh,
        scratch_types=[
            pltpu.VMEM((BK,), jnp.int32),
            pltpu.VMEM((BK,), jnp.int32),
            pltpu.VMEM((hpc * BK,), jnp.float32),
            pltpu.VMEM((BK, 128), jnp.float32),
            pltpu.VMEM((BK, 128), jnp.float32),
            pltpu.VMEM_SHARED((NP, 128), jnp.float32),
        ],
        compiler_params=pltpu.CompilerParams(needs_layout_passes=False),
    )


def _den2(den, nh):
    d = den.reshape(NS, NP, nh)
    d = jnp.transpose(d, (1, 2, 0))
    return d.reshape(NP, nh * NS)[:N]


def _sc_conv(src, dst, h, a_s, a_d, nh):
    hpc = nh // 2 if nh > 1 else 1
    w, den = _sc_den_fn(nh)(src, dst, a_s.reshape(N * nh),
                            a_d.reshape(N * nh))
    acc0 = _sc_num_fn(hpc, 0)(src, dst, h[:, 0:128], w)
    acc1 = _sc_num_fn(hpc, nh - hpc)(src, dst, h[:, 128:256], w)
    return jnp.concatenate([acc0[:N], acc1[:N]], axis=1), _den2(den, nh)


def _lin0_body(x_ref, w_ref, a_ref, h_out, a_out):
    h = jnp.dot(x_ref[:], w_ref[:], preferred_element_type=jnp.float32)
    h_out[:] = h
    a_out[:] = jnp.dot(h, a_ref[:], preferred_element_type=jnp.float32)


def _lin0(x, wcat, acat, blk=1000):
    return pl.pallas_call(
        _lin0_body,
        grid=(N // blk,),
        in_specs=[
            pl.BlockSpec((blk, IN_CH), lambda i: (i, 0)),
            pl.BlockSpec((IN_CH, 512), lambda i: (0, 0)),
            pl.BlockSpec((512, 16), lambda i: (0, 0)),
        ],
        out_specs=[
            pl.BlockSpec((blk, 512), lambda i: (i, 0)),
            pl.BlockSpec((blk, 16), lambda i: (i, 0)),
        ],
        out_shape=[
            jax.ShapeDtypeStruct((N, 512), jnp.float32),
            jax.ShapeDtypeStruct((N, 16), jnp.float32),
        ],
    )(x, wcat, acat)


def _leaky(x):
    return jnp.maximum(x, 0.2 * x)


def _combine0_body(numi_ref, numo_ref, deni_ref, deno_ref, h_ref, a_ref,
                   w1_ref, a1_ref, b0_ref, h1_out, a1_out):
    pieces = []
    for d, num_ref, den_ref in ((0, numi_ref, deni_ref),
                                (1, numo_ref, deno_ref)):
        num = num_ref[:]
        dpart = den_ref[:]
        for h in range(HEADS):
            numh = num[:, h * 64: (h + 1) * 64]
            den = jnp.sum(dpart[:, h * NS: (h + 1) * NS],
                          axis=1, keepdims=True)
            a_s = a_ref[:, d * 8 + h: d * 8 + h + 1]
            a_d = a_ref[:, d * 8 + 4 + h: d * 8 + 5 + h]
            wself = jnp.exp(_leaky(a_s + a_d))
            hh = h_ref[:, d * 256 + h * 64: d * 256 + (h + 1) * 64]
            bh = b0_ref[:, d * 256 + h * 64: d * 256 + (h + 1) * 64]
            pieces.append((numh + wself * hh) / (den + wself) + bh)
    x1 = jnp.maximum(jnp.concatenate(pieces, axis=1), 0.0)
    h1 = jnp.dot(x1, w1_ref[:], preferred_element_type=jnp.float32)
    h1_out[:] = h1
    a1_out[:] = jnp.dot(h1, a1_ref[:], preferred_element_type=jnp.float32)


def _combine0(numi, numo, deni2, deno2, hcat, acat, w1cat, a1cat, b0cat,
              blk=1000):
    return pl.pallas_call(
        _combine0_body,
        grid=(N // blk,),
        in_specs=[
            pl.BlockSpec((blk, 256), lambda i: (i, 0)),
            pl.BlockSpec((blk, 256), lambda i: (i, 0)),
            pl.BlockSpec((blk, HEADS * NS), lambda i: (i, 0)),
            pl.BlockSpec((blk, HEADS * NS), lambda i: (i, 0)),
            pl.BlockSpec((blk, 512), lambda i: (i, 0)),
            pl.BlockSpec((blk, 16), lambda i: (i, 0)),
            pl.BlockSpec((512, 512), lambda i: (0, 0)),
            pl.BlockSpec((512, 4), lambda i: (0, 0)),
            pl.BlockSpec((1, 512), lambda i: (0, 0)),
        ],
        out_specs=[
            pl.BlockSpec((blk, 512), lambda i: (i, 0)),
            pl.BlockSpec((blk, 4), lambda i: (i, 0)),
        ],
        out_shape=[
            jax.ShapeDtypeStruct((N, 512), jnp.float32),
            jax.ShapeDtypeStruct((N, 4), jnp.float32),
        ],
    )(numi, numo, deni2, deno2, hcat, acat, w1cat, a1cat, b0cat)


def _combine1_body(numi_ref, numo_ref, deni_ref, deno_ref, h1_ref, a1_ref,
                   b1_ref, out_ref):
    for d, num_ref, den_ref in ((0, numi_ref, deni_ref),
                                (1, numo_ref, deno_ref)):
        num = num_ref[:]
        den = jnp.sum(den_ref[:], axis=1, keepdims=True)
        a_s = a1_ref[:, d * 2: d * 2 + 1]
        a_d = a1_ref[:, d * 2 + 1: d * 2 + 2]
        wself = jnp.exp(_leaky(a_s + a_d))
        hh = h1_ref[:, d * 256: (d + 1) * 256]
        bh = b1_ref[:, d * 256: (d + 1) * 256]
        out_ref[:, d * 256: (d + 1) * 256] = (
            (num + wself * hh) / (den + wself) + bh)


def _combine1(numi, numo, deni2, deno2, h1cat, a1cat, b1cat, blk=1000):
    return pl.pallas_call(
        _combine1_body,
        grid=(N // blk,),
        in_specs=[
            pl.BlockSpec((blk, 256), lambda i: (i, 0)),
            pl.BlockSpec((blk, 256), lambda i: (i, 0)),
            pl.BlockSpec((blk, NS), lambda i: (i, 0)),
            pl.BlockSpec((blk, NS), lambda i: (i, 0)),
            pl.BlockSpec((blk, 512), lambda i: (i, 0)),
            pl.BlockSpec((blk, 4), lambda i: (i, 0)),
            pl.BlockSpec((1, 512), lambda i: (0, 0)),
        ],
        out_specs=pl.BlockSpec((blk, 512), lambda i: (i, 0)),
        out_shape=jax.ShapeDtypeStruct((N, 512), jnp.float32),
    )(numi, numo, deni2, deno2, h1cat, a1cat, b1cat)


def _att_cols(att):
    hh, cc = att.shape
    eye = jnp.eye(hh, dtype=att.dtype)
    return (att[:, :, None] * eye[:, None, :]).reshape(hh * cc, hh)


def kernel(x, edge_index, W0i, as0i, ad0i, b0i, W0o, as0o, ad0o, b0o,
           W1i, as1i, ad1i, b1i, W1o, as1o, ad1o, b1o):
    row, col = edge_index[0], edge_index[1]

    wcat0 = jnp.concatenate([W0i, W0o], axis=1)
    z8 = jnp.zeros((256, 8), jnp.float32)
    acat0 = jnp.concatenate([
        jnp.concatenate([_att_cols(as0i), _att_cols(ad0i), z8], axis=1),
        jnp.concatenate([z8, _att_cols(as0o), _att_cols(ad0o)], axis=1),
    ], axis=0)
    hcat0, acat = _lin0(x, wcat0, acat0)

    num_i, den_i = _sc_conv(row, col, hcat0[:, 0:256],
                            acat[:, 0:4], acat[:, 4:8], HEADS)
    num_o, den_o = _sc_conv(col, row, hcat0[:, 256:512],
                            acat[:, 8:12], acat[:, 12:16], HEADS)

    w1cat = jnp.concatenate([W1i, W1o], axis=1)
    z2 = jnp.zeros((256, 2), jnp.float32)
    a1cat = jnp.concatenate([
        jnp.concatenate([jnp.stack([as1i[0], ad1i[0]], axis=1), z2], axis=1),
        jnp.concatenate([z2, jnp.stack([as1o[0], ad1o[0]], axis=1)], axis=1),
    ], axis=0)
    b0cat = jnp.concatenate([b0i, b0o])[None, :]
    h1cat, a1 = _combine0(num_i, num_o, den_i, den_o, hcat0, acat,
                          w1cat, a1cat, b0cat)

    num1_i, den1_i = _sc_conv(row, col, h1cat[:, 0:256],
                              a1[:, 0:1], a1[:, 1:2], 1)
    num1_o, den1_o = _sc_conv(col, row, h1cat[:, 256:512],
                              a1[:, 2:3], a1[:, 3:4], 1)

    b1cat = jnp.concatenate([b1i, b1o])[None, :]
    return _combine1(num1_i, num1_o, den1_i, den1_o, h1cat, a1, b1cat)


# speedup vs baseline: 20.3658x; 1.7160x over previous
import functools

import jax
import jax.numpy as jnp
from jax import lax
from jax.experimental import pallas as pl
from jax.experimental.pallas import tpu as pltpu
from jax.experimental.pallas import tpu_sc as plsc

N = 10000
E = 320000
IN_CH = 128
HID = 256
HEADS = 4

NS = 16
BK = 80
NP = 10240
RPT = NP // NS
ET = E // NS
NB = ET // BK


def _sc_den_body(nh, src_hbm, dst_hbm, as_hbm, ad_hbm,
                 w_hbm, den_hbm, asv, adv, denv, idx_s, idx_d, wbuf):
    hd = 2 if nh > 1 else 1
    et = ET if nh > 1 else ET // 2
    nb = et // BK
    c = lax.axis_index("c")
    sid = lax.axis_index("s")
    lanes = lax.iota(jnp.int32, 16)
    zv = jnp.zeros((16,), jnp.float32)
    nph = NP * hd
    loff = jnp.bitwise_and(lanes, 3) * nph
    lgrp = lax.shift_right_logical(lanes, 2)

    pltpu.sync_copy(as_hbm.at[pl.ds(c * N * hd, N * hd)], asv)
    pltpu.sync_copy(ad_hbm.at[pl.ds(c * N * hd, N * hd)], adv)

    def _zero_den(k, _):
        denv[pl.ds(k * 16, 16)] = zv
        return 0
    lax.fori_loop(0, 4 * nph // 16, _zero_den, 0)

    woff = c * hd * E

    def _block(b, _):
        if nh > 1:
            base = sid * ET + b * BK
        else:
            base = (c * NS + sid) * et + b * BK
        pltpu.sync_copy(src_hbm.at[pl.ds(base, BK)], idx_s)
        pltpu.sync_copy(dst_hbm.at[pl.ds(base, BK)], idx_d)

        def _grp(g, _):
            sv = idx_s[pl.ds(g * 16, 16)]
            dv = idx_d[pl.ds(g * 16, 16)]
            if hd > 1:
                sv = sv * hd
                dv = dv * hd
            for j in range(hd):
                al = (plsc.load_gather(asv, [sv + j])
                      + plsc.load_gather(adv, [dv + j]))
                w = jnp.exp(jnp.maximum(al, 0.2 * al))
                wbuf[pl.ds(j * BK + g * 16, 16)] = w
                for m in range(4):
                    plsc.addupdate_scatter(denv, [dv + j + loff], w,
                                           mask=lgrp == m)
            return 0
        lax.fori_loop(0, BK // 16, _grp, 0)

        for j in range(hd):
            pltpu.sync_copy(wbuf.at[pl.ds(j * BK, BK)],
                            w_hbm.at[pl.ds(woff + j * E + base, BK)])
        return 0
    lax.fori_loop(0, nb, _block, 0)

    def _red(k, _):
        denv[pl.ds(k * 16, 16)] = (
            denv[pl.ds(k * 16, 16)]
            + denv[pl.ds(nph + k * 16, 16)]
            + denv[pl.ds(2 * nph + k * 16, 16)]
            + denv[pl.ds(3 * nph + k * 16, 16)])
        return 0
    lax.fori_loop(0, nph // 16, _red, 0)

    pltpu.sync_copy(denv.at[pl.ds(0, nph)],
                    den_hbm.at[pl.ds((c * NS + sid) * NP * hd, NP * hd)])


@functools.lru_cache(maxsize=None)
def _sc_den_fn(nh):
    mesh = plsc.VectorSubcoreMesh(core_axis_name="c", subcore_axis_name="s",
                                  num_cores=2)
    hd = 2 if nh > 1 else 1
    return pl.kernel(
        functools.partial(_sc_den_body, nh),
        out_type=[
            jax.ShapeDtypeStruct((nh * E,), jnp.float32),
            jax.ShapeDtypeStruct((2 * NS * NP * hd,), jnp.float32),
        ],
        mesh=mesh,
        scratch_types=[
            pltpu.VMEM((N * hd,), jnp.float32),
            pltpu.VMEM((N * hd,), jnp.float32),
            pltpu.VMEM((4 * NP * hd,), jnp.float32),
            pltpu.VMEM((BK,), jnp.int32),
            pltpu.VMEM((BK,), jnp.int32),
            pltpu.VMEM((hd * BK,), jnp.float32),
        ],
        compiler_params=pltpu.CompilerParams(needs_layout_passes=False),
    )


def _sc_num_body(hpc, src_hbm, dst_hbm, h2_hbm, w_hbm, acc_hbm,
                 idx_s, idx_d, wbuf, rows, stag, acc):
    cph = 128 // hpc
    c = lax.axis_index("c")
    sid = lax.axis_index("s")
    zv = jnp.zeros((16,), jnp.float32)
    woff = c * hpc * E

    def _zero_row(k, _):
        for g in range(128 // 16):
            stag[k, pl.ds(g * 16, 16)] = zv
        return 0
    lax.fori_loop(0, BK, _zero_row, 0)
    for i in range(RPT // BK):
        pltpu.sync_copy(stag, acc.at[pl.ds(sid * RPT + i * BK, BK)])
    plsc.subcore_barrier()

    def _block(b, _):
        base = sid * ET + b * BK
        pltpu.sync_copy(src_hbm.at[pl.ds(base, BK)], idx_s)
        pltpu.sync_copy(dst_hbm.at[pl.ds(base, BK)], idx_d)
        def _shift(g, _):
            idx_s[pl.ds(g * 16, 16)] = idx_s[pl.ds(g * 16, 16)] + c * N
            return 0
        lax.fori_loop(0, BK // 16, _shift, 0)
        pltpu.sync_copy(h2_hbm.at[idx_s], rows)

        for j in range(hpc):
            pltpu.sync_copy(w_hbm.at[pl.ds(woff + j * E + base, BK)],
                            wbuf.at[pl.ds(j * BK, BK)])

        def _grp(g, _):
            wvecs = [wbuf[pl.ds(j * BK + g * 16, 16)] for j in range(hpc)]
            for k16 in range(16):
                k = g * 16 + k16
                for j in range(hpc):
                    wv = jnp.full((16,), wvecs[j][k16])
                    for v in range(cph // 16):
                        col = j * cph + v * 16
                        stag[k, pl.ds(col, 16)] = (
                            rows[k, pl.ds(col, 16)] * wv)
            return 0
        lax.fori_loop(0, BK // 16, _grp, 0)

        pltpu.sync_copy(stag, acc.at[idx_d], add=True)
        return 0
    lax.fori_loop(0, NB, _block, 0)

    plsc.subcore_barrier()
    pltpu.sync_copy(acc.at[pl.ds(sid * RPT, RPT)],
                    acc_hbm.at[pl.ds(c * NP + sid * RPT, RPT)])


@functools.lru_cache(maxsize=None)
def _sc_num_fn(hpc):
    mesh = plsc.VectorSubcoreMesh(core_axis_name="c", subcore_axis_name="s",
                                  num_cores=2)
    return pl.kernel(
        functools.partial(_sc_num_body, hpc),
        out_type=jax.ShapeDtypeStruct((2 * NP, 128), jnp.float32),
        mesh=mesh,
        scratch_types=[
            pltpu.VMEM((BK,), jnp.int32),
            pltpu.VMEM((BK,), jnp.int32),
            pltpu.VMEM((hpc * BK,), jnp.float32),
            pltpu.VMEM((BK, 128), jnp.float32),
            pltpu.VMEM((BK, 128), jnp.float32),
            pltpu.VMEM_SHARED((NP, 128), jnp.float32),
        ],
        compiler_params=pltpu.CompilerParams(needs_layout_passes=False),
    )


def _den2(den, nh):
    if nh > 1:
        d = den.reshape(2, NS, NP, 2)
        d = jnp.transpose(d, (2, 0, 3, 1))
        return d.reshape(NP, nh * NS)[:N]
    d = den.reshape(2 * NS, NP)
    return jnp.transpose(d, (1, 0))[:N]


def _sc_conv(src, dst, h, a_s, a_d, nh):
    hpc = nh // 2 if nh > 1 else 1
    hd = 2 if nh > 1 else 1
    asf = jnp.concatenate([a_s[:, 0:hd].reshape(N * hd),
                           a_s[:, nh - hd:nh].reshape(N * hd)])
    adf = jnp.concatenate([a_d[:, 0:hd].reshape(N * hd),
                           a_d[:, nh - hd:nh].reshape(N * hd)])
    h2 = jnp.concatenate([h[:, 0:128], h[:, 128:256]], axis=0)
    w, den = _sc_den_fn(nh)(src, dst, asf, adf)
    acc = _sc_num_fn(hpc)(src, dst, h2, w)
    num = jnp.concatenate([acc[0:N], acc[NP:NP + N]], axis=1)
    return num, _den2(den, nh)


def _lin0_body(x_ref, w_ref, a_ref, h_out, a_out):
    h = jnp.dot(x_ref[:], w_ref[:], preferred_element_type=jnp.float32)
    h_out[:] = h
    a_out[:] = jnp.dot(h, a_ref[:], preferred_element_type=jnp.float32)


def _lin0(x, wcat, acat, blk=1000):
    return pl.pallas_call(
        _lin0_body,
        grid=(N // blk,),
        in_specs=[
            pl.BlockSpec((blk, IN_CH), lambda i: (i, 0)),
            pl.BlockSpec((IN_CH, 512), lambda i: (0, 0)),
            pl.BlockSpec((512, 16), lambda i: (0, 0)),
        ],
        out_specs=[
            pl.BlockSpec((blk, 512), lambda i: (i, 0)),
            pl.BlockSpec((blk, 16), lambda i: (i, 0)),
        ],
        out_shape=[
            jax.ShapeDtypeStruct((N, 512), jnp.float32),
            jax.ShapeDtypeStruct((N, 16), jnp.float32),
        ],
    )(x, wcat, acat)


def _leaky(x):
    return jnp.maximum(x, 0.2 * x)


def _combine0_body(numi_ref, numo_ref, deni_ref, deno_ref, h_ref, a_ref,
                   w1_ref, a1_ref, b0_ref, h1_out, a1_out):
    pieces = []
    for d, num_ref, den_ref in ((0, numi_ref, deni_ref),
                                (1, numo_ref, deno_ref)):
        num = num_ref[:]
        dpart = den_ref[:]
        for h in range(HEADS):
            numh = num[:, h * 64: (h + 1) * 64]
            den = jnp.sum(dpart[:, h * NS: (h + 1) * NS],
                          axis=1, keepdims=True)
            a_s = a_ref[:, d * 8 + h: d * 8 + h + 1]
            a_d = a_ref[:, d * 8 + 4 + h: d * 8 + 5 + h]
            wself = jnp.exp(_leaky(a_s + a_d))
            hh = h_ref[:, d * 256 + h * 64: d * 256 + (h + 1) * 64]
            bh = b0_ref[:, d * 256 + h * 64: d * 256 + (h + 1) * 64]
            pieces.append((numh + wself * hh) / (den + wself) + bh)
    x1 = jnp.maximum(jnp.concatenate(pieces, axis=1), 0.0)
    h1 = jnp.dot(x1, w1_ref[:], preferred_element_type=jnp.float32)
    h1_out[:] = h1
    a1_out[:] = jnp.dot(h1, a1_ref[:], preferred_element_type=jnp.float32)


def _combine0(numi, numo, deni2, deno2, hcat, acat, w1cat, a1cat, b0cat,
              blk=1000):
    return pl.pallas_call(
        _combine0_body,
        grid=(N // blk,),
        in_specs=[
            pl.BlockSpec((blk, 256), lambda i: (i, 0)),
            pl.BlockSpec((blk, 256), lambda i: (i, 0)),
            pl.BlockSpec((blk, HEADS * NS), lambda i: (i, 0)),
            pl.BlockSpec((blk, HEADS * NS), lambda i: (i, 0)),
            pl.BlockSpec((blk, 512), lambda i: (i, 0)),
            pl.BlockSpec((blk, 16), lambda i: (i, 0)),
            pl.BlockSpec((512, 512), lambda i: (0, 0)),
            pl.BlockSpec((512, 4), lambda i: (0, 0)),
            pl.BlockSpec((1, 512), lambda i: (0, 0)),
        ],
        out_specs=[
            pl.BlockSpec((blk, 512), lambda i: (i, 0)),
            pl.BlockSpec((blk, 4), lambda i: (i, 0)),
        ],
        out_shape=[
            jax.ShapeDtypeStruct((N, 512), jnp.float32),
            jax.ShapeDtypeStruct((N, 4), jnp.float32),
        ],
    )(numi, numo, deni2, deno2, hcat, acat, w1cat, a1cat, b0cat)


def _combine1_body(numi_ref, numo_ref, deni_ref, deno_ref, h1_ref, a1_ref,
                   b1_ref, out_ref):
    for d, num_ref, den_ref in ((0, numi_ref, deni_ref),
                                (1, numo_ref, deno_ref)):
        num = num_ref[:]
        den = jnp.sum(den_ref[:], axis=1, keepdims=True)
        a_s = a1_ref[:, d * 2: d * 2 + 1]
        a_d = a1_ref[:, d * 2 + 1: d * 2 + 2]
        wself = jnp.exp(_leaky(a_s + a_d))
        hh = h1_ref[:, d * 256: (d + 1) * 256]
        bh = b1_ref[:, d * 256: (d + 1) * 256]
        out_ref[:, d * 256: (d + 1) * 256] = (
            (num + wself * hh) / (den + wself) + bh)


def _combine1(numi, numo, deni2, deno2, h1cat, a1cat, b1cat, blk=1000):
    return pl.pallas_call(
        _combine1_body,
        grid=(N // blk,),
        in_specs=[
            pl.BlockSpec((blk, 256), lambda i: (i, 0)),
            pl.BlockSpec((blk, 256), lambda i: (i, 0)),
            pl.BlockSpec((blk, 2 * NS), lambda i: (i, 0)),
            pl.BlockSpec((blk, 2 * NS), lambda i: (i, 0)),
            pl.BlockSpec((blk, 512), lambda i: (i, 0)),
            pl.BlockSpec((blk, 4), lambda i: (i, 0)),
            pl.BlockSpec((1, 512), lambda i: (0, 0)),
        ],
        out_specs=pl.BlockSpec((blk, 512), lambda i: (i, 0)),
        out_shape=jax.ShapeDtypeStruct((N, 512), jnp.float32),
    )(numi, numo, deni2, deno2, h1cat, a1cat, b1cat)


def _att_cols(att):
    hh, cc = att.shape
    eye = jnp.eye(hh, dtype=att.dtype)
    return (att[:, :, None] * eye[:, None, :]).reshape(hh * cc, hh)


def kernel(x, edge_index, W0i, as0i, ad0i, b0i, W0o, as0o, ad0o, b0o,
           W1i, as1i, ad1i, b1i, W1o, as1o, ad1o, b1o):
    row, col = edge_index[0], edge_index[1]

    wcat0 = jnp.concatenate([W0i, W0o], axis=1)
    z8 = jnp.zeros((256, 8), jnp.float32)
    acat0 = jnp.concatenate([
        jnp.concatenate([_att_cols(as0i), _att_cols(ad0i), z8], axis=1),
        jnp.concatenate([z8, _att_cols(as0o), _att_cols(ad0o)], axis=1),
    ], axis=0)
    hcat0, acat = _lin0(x, wcat0, acat0)

    num_i, den_i = _sc_conv(row, col, hcat0[:, 0:256],
                            acat[:, 0:4], acat[:, 4:8], HEADS)
    num_o, den_o = _sc_conv(col, row, hcat0[:, 256:512],
                            acat[:, 8:12], acat[:, 12:16], HEADS)

    w1cat = jnp.concatenate([W1i, W1o], axis=1)
    z2 = jnp.zeros((256, 2), jnp.float32)
    a1cat = jnp.concatenate([
        jnp.concatenate([jnp.stack([as1i[0], ad1i[0]], axis=1), z2], axis=1),
        jnp.concatenate([z2, jnp.stack([as1o[0], ad1o[0]], axis=1)], axis=1),
    ], axis=0)
    b0cat = jnp.concatenate([b0i, b0o])[None, :]
    h1cat, a1 = _combine0(num_i, num_o, den_i, den_o, hcat0, acat,
                          w1cat, a1cat, b0cat)

    num1_i, den1_i = _sc_conv(row, col, h1cat[:, 0:256],
                              a1[:, 0:1], a1[:, 1:2], 1)
    num1_o, den1_o = _sc_conv(col, row, h1cat[:, 256:512],
                              a1[:, 2:3], a1[:, 3:4], 1)

    b1cat = jnp.concatenate([b1i, b1o])[None, :]
    return _combine1(num1_i, num1_o, den1_i, den1_o, h1cat, a1, b1cat)
